# initial kernel scaffold (unmeasured)
import jax
import jax.numpy as jnp
from jax import lax
from jax.experimental import pallas as pl
from jax.experimental.pallas import tpu as pltpu


def kernel(
    x,
):
    def body(*refs):
        pass

    out_shape = jax.ShapeDtypeStruct(..., jnp.float32)
    return pl.pallas_call(body, out_shape=out_shape)(...)


# baseline (device time: 116737 ns/iter reference)
import jax
import jax.numpy as jnp
from jax import lax
from jax.experimental import pallas as pl
from jax.experimental.pallas import tpu as pltpu


def kernel(x):
    _, M, N2 = x.shape
    N = N2 // 2

    def body(x_ref, out_ref, send_buf, recv_buf, send_sem, recv_sem):
        my_x = lax.axis_index("x")
        my_y = lax.axis_index("y")
        peer = (1 - my_x, my_y)

        barrier_sem = pltpu.get_barrier_semaphore()
        pl.semaphore_signal(
            barrier_sem, inc=1, device_id=peer,
            device_id_type=pl.DeviceIdType.MESH,
        )
        pl.semaphore_wait(barrier_sem, 1)

        peer_col = (1 - my_x) * N
        send_buf[...] = x_ref[0, :, pl.ds(peer_col, N)].astype(jnp.bfloat16)
        rdma = pltpu.make_async_remote_copy(
            src_ref=send_buf,
            dst_ref=recv_buf,
            send_sem=send_sem,
            recv_sem=recv_sem,
            device_id=peer,
            device_id_type=pl.DeviceIdType.MESH,
        )
        rdma.start()
        rdma.wait()

        my_col = my_x * N
        out_ref[...] = (
            x_ref[0, :, pl.ds(my_col, N)]
            + recv_buf[...].astype(jnp.float32)
        ).astype(jnp.bfloat16)

    return pl.pallas_call(
        body,
        out_shape=jax.ShapeDtypeStruct((M, N), jnp.bfloat16),
        in_specs=[pl.BlockSpec(memory_space=pltpu.VMEM)],
        out_specs=pl.BlockSpec(memory_space=pltpu.VMEM),
        scratch_shapes=[
            pltpu.VMEM((M, N), jnp.bfloat16),
            pltpu.VMEM((M, N), jnp.bfloat16),
            pltpu.SemaphoreType.DMA,
            pltpu.SemaphoreType.DMA,
        ],
        compiler_params=pltpu.CompilerParams(
            collective_id=0, vmem_limit_bytes=100 * 1024 * 1024
        ),
    )(x)


# device time: 79183 ns/iter; 1.4743x vs baseline; 1.4743x over previous
import functools

import jax
import jax.numpy as jnp
from jax import lax
from jax.experimental import pallas as pl
from jax.experimental.pallas import tpu as pltpu

K = 8


def kernel(x):
    _, M, N2 = x.shape
    N = N2 // 2
    H = M // 2
    R = H // K

    def body(
        x_ref,
        out_ref,
        xsend_buf,
        xrecv_buf,
        yrecv_buf,
        xsend_sems,
        xrecv_sems,
        ysend_sems,
        yrecv_sems,
    ):
        my_x = lax.axis_index("x")
        my_y = lax.axis_index("y")
        x_peer = (1 - my_x, my_y)
        y_peer = (my_x, 1 - my_y)

        cols_mine = my_x * N
        cols_peer = (1 - my_x) * N
        rows_mine = my_y * H
        rows_other = (1 - my_y) * H

        barrier_sem = pltpu.get_barrier_semaphore()
        for nbr in (x_peer, y_peer):
            pl.semaphore_signal(
                barrier_sem, inc=1, device_id=nbr,
                device_id_type=pl.DeviceIdType.MESH,
            )
        pl.semaphore_wait(barrier_sem, 2)

        def x_rdma(c):
            return pltpu.make_async_remote_copy(
                src_ref=xsend_buf.at[c],
                dst_ref=xrecv_buf.at[c],
                send_sem=xsend_sems.at[c],
                recv_sem=xrecv_sems.at[c],
                device_id=x_peer,
                device_id_type=pl.DeviceIdType.MESH,
            )

        def y_rdma(c):
            return pltpu.make_async_remote_copy(
                src_ref=xrecv_buf.at[c],
                dst_ref=yrecv_buf.at[c],
                send_sem=ysend_sems.at[c],
                recv_sem=yrecv_sems.at[c],
                device_id=y_peer,
                device_id_type=pl.DeviceIdType.MESH,
            )

        for c in range(K):
            xsend_buf[c] = x_ref[
                0, pl.ds(rows_mine + c * R, R), pl.ds(cols_peer, N)
            ].astype(jnp.bfloat16)
            x_rdma(c).start()

        for c in range(K):
            x_rdma(c).wait_recv()
            y_rdma(c).start()
            out_ref[pl.ds(rows_mine + c * R, R), :] = (
                x_ref[0, pl.ds(rows_mine + c * R, R), pl.ds(cols_mine, N)]
                + xrecv_buf[c].astype(jnp.float32)
            ).astype(jnp.bfloat16)

        for c in range(K):
            y_rdma(c).wait_recv()
            out_ref[pl.ds(rows_other + c * R, R), :] = (
                x_ref[0, pl.ds(rows_other + c * R, R), pl.ds(cols_mine, N)]
                + yrecv_buf[c].astype(jnp.float32)
            ).astype(jnp.bfloat16)

        for c in range(K):
            x_rdma(c).wait_send()
            y_rdma(c).wait_send()

        @functools.partial(
            pl.run_scoped, exit_sem=pltpu.SemaphoreType.REGULAR
        )
        def _(exit_sem):
            for nbr in (x_peer, y_peer):
                pl.semaphore_signal(
                    exit_sem, inc=1, device_id=nbr,
                    device_id_type=pl.DeviceIdType.MESH,
                )
            pl.semaphore_wait(exit_sem, 2)

    return pl.pallas_call(
        body,
        out_shape=jax.ShapeDtypeStruct((M, N), jnp.bfloat16),
        in_specs=[pl.BlockSpec(memory_space=pltpu.VMEM)],
        out_specs=pl.BlockSpec(memory_space=pltpu.VMEM),
        scratch_shapes=[
            pltpu.VMEM((K, R, N), jnp.bfloat16),
            pltpu.VMEM((K, R, N), jnp.bfloat16),
            pltpu.VMEM((K, R, N), jnp.bfloat16),
            pltpu.SemaphoreType.DMA((K,)),
            pltpu.SemaphoreType.DMA((K,)),
            pltpu.SemaphoreType.DMA((K,)),
            pltpu.SemaphoreType.DMA((K,)),
        ],
        compiler_params=pltpu.CompilerParams(
            collective_id=0, vmem_limit_bytes=100 * 1024 * 1024
        ),
    )(x)


# device time: 66800 ns/iter; 1.7476x vs baseline; 1.1854x over previous
import functools

import jax
import jax.numpy as jnp
from jax import lax
from jax.experimental import pallas as pl
from jax.experimental.pallas import tpu as pltpu

K = 8


def kernel(x):
    _, M, N2 = x.shape
    N = N2 // 2
    H = M // 2
    R = H // K

    def body(
        x_hbm,
        out_hbm,
        fsend,
        fa,
        fb,
        xsend_buf,
        xrecv_buf,
        yrecv_buf,
        oa,
        ob,
        fsend_dsem,
        fa_dsem,
        fb_dsem,
        oa_dsem,
        ob_dsem,
        xsend_sems,
        xrecv_sems,
        ysend_sems,
        yrecv_sems,
    ):
        my_x = lax.axis_index("x")
        my_y = lax.axis_index("y")
        x_peer = (1 - my_x, my_y)
        y_peer = (my_x, 1 - my_y)

        cols_mine = my_x * N
        cols_peer = (1 - my_x) * N
        rows_mine = my_y * H
        rows_other = (1 - my_y) * H

        def fsend_dma(c):
            return pltpu.make_async_copy(
                x_hbm.at[0, pl.ds(rows_mine + c * R, R), pl.ds(cols_peer, N)],
                fsend.at[c], fsend_dsem.at[c])

        def fa_dma(c):
            return pltpu.make_async_copy(
                x_hbm.at[0, pl.ds(rows_mine + c * R, R), pl.ds(cols_mine, N)],
                fa.at[c], fa_dsem.at[c])

        def fb_dma(c):
            return pltpu.make_async_copy(
                x_hbm.at[0, pl.ds(rows_other + c * R, R), pl.ds(cols_mine, N)],
                fb.at[c], fb_dsem.at[c])

        def oa_dma(c):
            return pltpu.make_async_copy(
                oa.at[c], out_hbm.at[pl.ds(rows_mine + c * R, R), :],
                oa_dsem.at[c])

        def ob_dma(c):
            return pltpu.make_async_copy(
                ob.at[c], out_hbm.at[pl.ds(rows_other + c * R, R), :],
                ob_dsem.at[c])

        def x_rdma(c):
            return pltpu.make_async_remote_copy(
                src_ref=xsend_buf.at[c],
                dst_ref=xrecv_buf.at[c],
                send_sem=xsend_sems.at[c],
                recv_sem=xrecv_sems.at[c],
                device_id=x_peer,
                device_id_type=pl.DeviceIdType.MESH,
            )

        def y_rdma(c):
            return pltpu.make_async_remote_copy(
                src_ref=xrecv_buf.at[c],
                dst_ref=yrecv_buf.at[c],
                send_sem=ysend_sems.at[c],
                recv_sem=yrecv_sems.at[c],
                device_id=y_peer,
                device_id_type=pl.DeviceIdType.MESH,
            )

        for c in range(K):
            fsend_dma(c).start()
        for c in range(K):
            fa_dma(c).start()
        for c in range(K):
            fb_dma(c).start()

        barrier_sem = pltpu.get_barrier_semaphore()
        for nbr in (x_peer, y_peer):
            pl.semaphore_signal(
                barrier_sem, inc=1, device_id=nbr,
                device_id_type=pl.DeviceIdType.MESH,
            )
        pl.semaphore_wait(barrier_sem, 2)

        for c in range(K):
            fsend_dma(c).wait()
            xsend_buf[c] = fsend[c].astype(jnp.bfloat16)
            x_rdma(c).start()

        for c in range(K):
            x_rdma(c).wait_recv()
            y_rdma(c).start()
            fa_dma(c).wait()
            oa[c] = (fa[c] + xrecv_buf[c].astype(jnp.float32)).astype(
                jnp.bfloat16)
            oa_dma(c).start()

        for c in range(K):
            y_rdma(c).wait_recv()
            fb_dma(c).wait()
            ob[c] = (fb[c] + yrecv_buf[c].astype(jnp.float32)).astype(
                jnp.bfloat16)
            ob_dma(c).start()

        for c in range(K):
            x_rdma(c).wait_send()
            y_rdma(c).wait_send()
            oa_dma(c).wait()
            ob_dma(c).wait()

        @functools.partial(
            pl.run_scoped, exit_sem=pltpu.SemaphoreType.REGULAR
        )
        def _(exit_sem):
            for nbr in (x_peer, y_peer):
                pl.semaphore_signal(
                    exit_sem, inc=1, device_id=nbr,
                    device_id_type=pl.DeviceIdType.MESH,
                )
            pl.semaphore_wait(exit_sem, 2)

    return pl.pallas_call(
        body,
        out_shape=jax.ShapeDtypeStruct((M, N), jnp.bfloat16),
        in_specs=[pl.BlockSpec(memory_space=pl.ANY)],
        out_specs=pl.BlockSpec(memory_space=pl.ANY),
        scratch_shapes=[
            pltpu.VMEM((K, R, N), jnp.float32),
            pltpu.VMEM((K, R, N), jnp.float32),
            pltpu.VMEM((K, R, N), jnp.float32),
            pltpu.VMEM((K, R, N), jnp.bfloat16),
            pltpu.VMEM((K, R, N), jnp.bfloat16),
            pltpu.VMEM((K, R, N), jnp.bfloat16),
            pltpu.VMEM((K, R, N), jnp.bfloat16),
            pltpu.VMEM((K, R, N), jnp.bfloat16),
            pltpu.SemaphoreType.DMA((K,)),
            pltpu.SemaphoreType.DMA((K,)),
            pltpu.SemaphoreType.DMA((K,)),
            pltpu.SemaphoreType.DMA((K,)),
            pltpu.SemaphoreType.DMA((K,)),
            pltpu.SemaphoreType.DMA((K,)),
            pltpu.SemaphoreType.DMA((K,)),
            pltpu.SemaphoreType.DMA((K,)),
            pltpu.SemaphoreType.DMA((K,)),
        ],
        compiler_params=pltpu.CompilerParams(
            collective_id=0, vmem_limit_bytes=100 * 1024 * 1024
        ),
    )(x)
